# baseline (speedup 1.0000x reference)
import functools

import jax
import jax.numpy as jnp
from jax import lax
from jax.experimental import pallas as pl
from jax.experimental.pallas import tpu as pltpu
from jax.experimental.pallas import tpu_sc as plsc

_LANES = 128
_NCLS = 29
_HBINS = 32
_NSUB = 16


def _conv_vote_body(x_ref, w_ref, b_ref, votes_ref):
    logits = lax.dot_general(
        x_ref[...], w_ref[...],
        dimension_numbers=(((1,), (1,)), ((), ())),
        preferred_element_type=jnp.float32,
    ) + b_ref[...]
    lane = jax.lax.broadcasted_iota(jnp.int32, logits.shape, 1)
    rowmax = jnp.max(logits, axis=1, keepdims=True)
    vote = jnp.min(jnp.where(logits == rowmax, lane, _LANES), axis=1,
                   keepdims=True)
    votes_ref[...] = vote.reshape(votes_ref.shape)


def _sc_hist_body(chunk, votes_hbm, out_hbm, votes_v, hist_v, gath_v,
                  shared, out_v):
    cid = lax.axis_index("c")
    sid = lax.axis_index("s")

    @pl.when(cid == 0)
    def _core0():
        pltpu.sync_copy(votes_hbm.at[pl.ds(sid * chunk, chunk)], votes_v)
        zeros16 = jnp.zeros((16,), jnp.float32)
        hist_v[pl.ds(0, 16)] = zeros16
        hist_v[pl.ds(16, 16)] = zeros16
        ones16 = jnp.ones((16,), jnp.float32)

        def body(j, carry):
            v = votes_v[pl.ds(pl.multiple_of(j * 16, 16), 16)]
            plsc.addupdate_scatter(hist_v, [v], ones16)
            return carry

        lax.fori_loop(0, chunk // 16, body, 0)
        pltpu.sync_copy(hist_v, shared.at[sid])
        plsc.subcore_barrier()

        @pl.when(sid == 0)
        def _finalize():
            pltpu.sync_copy(shared, gath_v)
            h0 = jnp.zeros((16,), jnp.float32)
            h1 = jnp.zeros((16,), jnp.float32)
            for j in range(_NSUB):
                h0 = h0 + gath_v[j, pl.ds(0, 16)]
                h1 = h1 + gath_v[j, pl.ds(16, 16)]
            m = jnp.maximum(jnp.max(h0), jnp.max(h1))
            iota = lax.iota(jnp.int32, 16)
            w0 = jnp.min(jnp.where(h0 == m, iota, _LANES))
            w1 = jnp.min(jnp.where(h1 == m, iota + 16, _LANES))
            winner = jnp.minimum(w0, w1)
            out_v[pl.ds(0, 16)] = (iota == winner).astype(jnp.float32)
            out_v[pl.ds(16, 16)] = ((iota + 16) == winner).astype(jnp.float32)
            pltpu.sync_copy(out_v, out_hbm)


def kernel(x, W, b):
    _, T, C = x.shape
    xs = x.reshape(T, C)
    b2 = b.reshape(1, _NCLS)
    Tb = 2048
    votes = pl.pallas_call(
        _conv_vote_body,
        grid=(T // Tb,),
        in_specs=[
            pl.BlockSpec((Tb, C), lambda i: (i, 0)),
            pl.BlockSpec((_NCLS, C), lambda i: (0, 0)),
            pl.BlockSpec((1, _NCLS), lambda i: (0, 0)),
        ],
        out_specs=pl.BlockSpec((Tb // _LANES, _LANES), lambda i: (i, 0)),
        out_shape=jax.ShapeDtypeStruct((T // _LANES, _LANES), jnp.int32),
    )(xs, W, b2)

    chunk = T // _NSUB
    mesh = plsc.VectorSubcoreMesh(core_axis_name="c", subcore_axis_name="s")
    sc_hist = functools.partial(
        pl.kernel,
        out_type=jax.ShapeDtypeStruct((_HBINS,), jnp.float32),
        mesh=mesh,
        compiler_params=pltpu.CompilerParams(needs_layout_passes=False),
        scratch_types=[
            pltpu.VMEM((chunk,), jnp.int32),
            pltpu.VMEM((_HBINS,), jnp.float32),
            pltpu.VMEM((_NSUB, _HBINS), jnp.float32),
            pltpu.VMEM_SHARED((_NSUB, _HBINS), jnp.float32),
            pltpu.VMEM((_HBINS,), jnp.float32),
        ],
    )(functools.partial(_sc_hist_body, chunk))
    hist_onehot = sc_hist(votes.reshape(T))
    return hist_onehot[:_NCLS].reshape(1, _NCLS)


# speedup vs baseline: 1.0362x; 1.0362x over previous
import functools

import jax
import jax.numpy as jnp
from jax import lax
from jax.experimental import pallas as pl
from jax.experimental.pallas import tpu as pltpu
from jax.experimental.pallas import tpu_sc as plsc

_LANES = 128
_NCLS = 29
_HBINS = 32
_NSUB = 16


def _conv_vote_body(x_ref, w_ref, b_ref, votes_ref):
    logits = lax.dot_general(
        x_ref[...], w_ref[...],
        dimension_numbers=(((1,), (1,)), ((), ())),
        preferred_element_type=jnp.float32,
    ) + b_ref[...]
    lane = jax.lax.broadcasted_iota(jnp.int32, logits.shape, 1)
    rowmax = jnp.max(logits, axis=1, keepdims=True)
    vote = jnp.min(jnp.where(logits == rowmax, lane, _LANES), axis=1,
                   keepdims=True)
    votes_ref[...] = vote.reshape(votes_ref.shape)


def _sc_hist_body(chunk, votes_hbm, out_hbm, votes_v, hist_v, gath_v,
                  shared, out_v):
    cid = lax.axis_index("c")
    sid = lax.axis_index("s")

    @pl.when(cid == 0)
    def _core0():
        pltpu.sync_copy(votes_hbm.at[pl.ds(sid * chunk, chunk)], votes_v)
        zeros16 = jnp.zeros((16,), jnp.float32)
        hist_v[pl.ds(0, 16)] = zeros16
        hist_v[pl.ds(16, 16)] = zeros16
        ones16 = jnp.ones((16,), jnp.float32)

        def body(j, carry):
            v = votes_v[pl.ds(pl.multiple_of(j * 16, 16), 16)]
            plsc.addupdate_scatter(hist_v, [v], ones16)
            return carry

        lax.fori_loop(0, chunk // 16, body, 0)
        pltpu.sync_copy(hist_v, shared.at[sid])
        plsc.subcore_barrier()

        @pl.when(sid == 0)
        def _finalize():
            pltpu.sync_copy(shared, gath_v)
            h0 = jnp.zeros((16,), jnp.float32)
            h1 = jnp.zeros((16,), jnp.float32)
            for j in range(_NSUB):
                h0 = h0 + gath_v[j, pl.ds(0, 16)]
                h1 = h1 + gath_v[j, pl.ds(16, 16)]
            m = jnp.maximum(jnp.max(h0), jnp.max(h1))
            iota = lax.iota(jnp.int32, 16)
            w0 = jnp.min(jnp.where(h0 == m, iota, _LANES))
            w1 = jnp.min(jnp.where(h1 == m, iota + 16, _LANES))
            winner = jnp.minimum(w0, w1)
            out_v[pl.ds(0, 16)] = (iota == winner).astype(jnp.float32)
            out_v[pl.ds(16, 16)] = ((iota + 16) == winner).astype(jnp.float32)
            pltpu.sync_copy(out_v.at[pl.ds(0, _NCLS)], out_hbm.at[0])


def kernel(x, W, b):
    _, T, C = x.shape
    xs = x.reshape(T, C)
    b2 = b.reshape(1, _NCLS)
    Tb = 2048
    votes = pl.pallas_call(
        _conv_vote_body,
        grid=(T // Tb,),
        in_specs=[
            pl.BlockSpec((Tb, C), lambda i: (i, 0)),
            pl.BlockSpec((_NCLS, C), lambda i: (0, 0)),
            pl.BlockSpec((1, _NCLS), lambda i: (0, 0)),
        ],
        out_specs=pl.BlockSpec((Tb // _LANES, _LANES), lambda i: (i, 0)),
        out_shape=jax.ShapeDtypeStruct((T // _LANES, _LANES), jnp.int32),
    )(xs, W, b2)

    chunk = T // _NSUB
    mesh = plsc.VectorSubcoreMesh(core_axis_name="c", subcore_axis_name="s",
                                  num_cores=1)
    sc_hist = functools.partial(
        pl.kernel,
        out_type=jax.ShapeDtypeStruct((1, _NCLS), jnp.float32),
        mesh=mesh,
        compiler_params=pltpu.CompilerParams(needs_layout_passes=False),
        scratch_types=[
            pltpu.VMEM((chunk,), jnp.int32),
            pltpu.VMEM((_HBINS,), jnp.float32),
            pltpu.VMEM((_NSUB, _HBINS), jnp.float32),
            pltpu.VMEM_SHARED((_NSUB, _HBINS), jnp.float32),
            pltpu.VMEM((_HBINS,), jnp.float32),
        ],
    )(functools.partial(_sc_hist_body, chunk))
    return sc_hist(votes.reshape(T))


# speedup vs baseline: 1.3815x; 1.3332x over previous
import jax
import jax.numpy as jnp
from jax import lax
from jax.experimental import pallas as pl
from jax.experimental.pallas import tpu as pltpu

_LANES = 128
_NCLS = 29


def _fused_body(x_ref, w_ref, b_ref, out_ref, acc_ref):
    i = pl.program_id(0)
    n = pl.num_programs(0)

    @pl.when(i == 0)
    def _init():
        acc_ref[...] = jnp.zeros_like(acc_ref)

    logits = lax.dot_general(
        x_ref[...], w_ref[...],
        dimension_numbers=(((1,), (1,)), ((), ())),
        preferred_element_type=jnp.float32,
    ) + b_ref[...]
    lane = jax.lax.broadcasted_iota(jnp.int32, logits.shape, 1)
    rowmax = jnp.max(logits, axis=1, keepdims=True)
    vote = jnp.min(jnp.where(logits == rowmax, lane, _LANES), axis=1,
                   keepdims=True)
    onehot = (lane == vote).astype(jnp.float32)
    acc_ref[...] += jnp.sum(onehot, axis=0, keepdims=True)

    @pl.when(i == n - 1)
    def _fin():
        counts = acc_ref[...]
        cmax = jnp.max(counts)
        l1 = jax.lax.broadcasted_iota(jnp.int32, counts.shape, 1)
        winner = jnp.min(jnp.where(counts == cmax, l1, _LANES))
        out_ref[...] = (l1 == winner).astype(jnp.float32)


def kernel(x, W, b):
    _, T, C = x.shape
    xs = x.reshape(T, C)
    b2 = b.reshape(1, _NCLS)
    Tb = 2048
    return pl.pallas_call(
        _fused_body,
        grid=(T // Tb,),
        in_specs=[
            pl.BlockSpec((Tb, C), lambda i: (i, 0)),
            pl.BlockSpec((_NCLS, C), lambda i: (0, 0)),
            pl.BlockSpec((1, _NCLS), lambda i: (0, 0)),
        ],
        out_specs=pl.BlockSpec((1, _NCLS), lambda i: (0, 0)),
        out_shape=jax.ShapeDtypeStruct((1, _NCLS), jnp.float32),
        scratch_shapes=[pltpu.VMEM((1, _NCLS), jnp.float32)],
    )(xs, W, b2)
